# baseline (speedup 1.0000x reference)
import functools

import jax
import jax.numpy as jnp
from jax import lax
from jax.experimental import pallas as pl
from jax.experimental.pallas import tpu as pltpu
from jax.experimental.pallas import tpu_sc as plsc

_N = 10000
_E = 320000
_H = 8
_HID = 8
_NC = 40
_BLK = 2000

_NCORE = 2
_NSUB = 16
_NW = _NCORE * _NSUB
_EW = _E // _NW
_C = 40
_CPW = _EW // _C
_NBUF = 5
_UNROLL = 10
_DEN = 16


def _leaky(v):
    return jnp.where(v >= 0, v, 0.2 * v)


def _dot(a, b):
    return jnp.dot(a, b, preferred_element_type=jnp.float32)


def _lin_body(h_ref, wl_ref, wr_ref, xl_ref, xr_ref):
    h = h_ref[...]
    xl_ref[...] = _dot(h, wl_ref[...])
    xr_ref[...] = _dot(h, wr_ref[...])


def _lin(h, Wl, Wr):
    n, din = h.shape
    w = Wl.shape[1]
    return pl.pallas_call(
        _lin_body,
        grid=(n // _BLK,),
        in_specs=[
            pl.BlockSpec((_BLK, din), lambda i: (i, 0)),
            pl.BlockSpec((din, w), lambda i: (0, 0)),
            pl.BlockSpec((din, w), lambda i: (0, 0)),
        ],
        out_specs=[
            pl.BlockSpec((_BLK, w), lambda i: (i, 0)),
            pl.BlockSpec((_BLK, w), lambda i: (i, 0)),
        ],
        out_shape=[jax.ShapeDtypeStruct((n, w), jnp.float32)] * 2,
    )(h, Wl, Wr)


def _mid_body(acc_ref, xl_ref, xr_ref, hprev_ref, a_ref, k_ref, b_ref,
              r_ref, rb_ref, h_ref):
    xl = xl_ref[...]
    xr = xr_ref[...]
    w = xl.shape[1]
    exs = jnp.exp(_dot(_leaky(xl + xr), a_ref[...]))
    acc = acc_ref[0] + acc_ref[1]
    num = acc[:, :w] + _dot(exs, k_ref[...]) * xl
    den = acc[:, w:w + _H] + exs
    denx = _dot(den, k_ref[...]) + 1e-16
    gat = num / denx + b_ref[...]
    h_ref[...] = _leaky(gat + _dot(hprev_ref[...], r_ref[...]) + rb_ref[...])


def _mid(acc, xl, xr, hprev, A, K, b, R, rb):
    n, w = xl.shape
    din = hprev.shape[1]
    f = w + _DEN
    return pl.pallas_call(
        _mid_body,
        grid=(n // _BLK,),
        in_specs=[
            pl.BlockSpec((2, _BLK, f), lambda i: (0, i, 0)),
            pl.BlockSpec((_BLK, w), lambda i: (i, 0)),
            pl.BlockSpec((_BLK, w), lambda i: (i, 0)),
            pl.BlockSpec((_BLK, din), lambda i: (i, 0)),
            pl.BlockSpec((w, _H), lambda i: (0, 0)),
            pl.BlockSpec((_H, w), lambda i: (0, 0)),
            pl.BlockSpec((1, w), lambda i: (0, 0)),
            pl.BlockSpec((din, w), lambda i: (0, 0)),
            pl.BlockSpec((1, w), lambda i: (0, 0)),
        ],
        out_specs=pl.BlockSpec((_BLK, w), lambda i: (i, 0)),
        out_shape=jax.ShapeDtypeStruct((n, w), jnp.float32),
    )(acc, xl, xr, hprev, A, K, b, R, rb)


def _final_body(*refs):
    accs = refs[0:4]
    xls = refs[4:8]
    xrs = refs[8:12]
    aas = refs[12:16]
    k_ref, s_ref, b_ref, out_ref = refs[16:]
    w = xls[0].shape[1]

    tot = None
    for acc_ref, xl_ref, xr_ref, a_ref in zip(accs, xls, xrs, aas):
        xl = xl_ref[...]
        exs = jnp.exp(_dot(_leaky(xl + xr_ref[...]), a_ref[...]))
        acc = acc_ref[0] + acc_ref[1]
        num = acc[:, :w] + _dot(exs, k_ref[...]) * xl
        den = acc[:, w:w + 2] + exs
        gat = num / (_dot(den, k_ref[...]) + 1e-16)
        part = _dot(gat, s_ref[...])
        tot = part if tot is None else tot + part
    out = tot * 0.125 + b_ref[...]
    mx = jnp.max(out, axis=1, keepdims=True)
    lse = jnp.log(jnp.sum(jnp.exp(out - mx), axis=1, keepdims=True)) + mx
    out_ref[...] = out - lse


def _final(accs, xls, xrs, As, K, S, b5):
    n, w = xls[0].shape
    f = w + _DEN
    acc_spec = pl.BlockSpec((2, _BLK, f), lambda i: (0, i, 0))
    row_spec = pl.BlockSpec((_BLK, w), lambda i: (i, 0))
    a_spec = pl.BlockSpec((w, 2), lambda i: (0, 0))
    return pl.pallas_call(
        _final_body,
        grid=(n // _BLK,),
        in_specs=([acc_spec] * 4 + [row_spec] * 8 + [a_spec] * 4 + [
            pl.BlockSpec((2, w), lambda i: (0, 0)),
            pl.BlockSpec((w, _NC), lambda i: (0, 0)),
            pl.BlockSpec((1, _NC), lambda i: (0, 0)),
        ]),
        out_specs=pl.BlockSpec((_BLK, _NC), lambda i: (i, 0)),
        out_shape=jax.ShapeDtypeStruct((n, _NC), jnp.float32),
    )(*accs, *xls, *xrs, *As, K, S, b5)


_GD = lax.GatherDimensionNumbers(
    offset_dims=(), collapsed_slice_dims=(0,), start_index_map=(0,))


def _take(v, idx):
    return lax.gather(v, idx[:, None], _GD, (1,),
                      mode=lax.GatherScatterMode.PROMISE_IN_BOUNDS)


def _ec_ch8(gl, gr, atts, vals, es):
    iota = lax.iota(jnp.int32, 16)
    dsel = (iota & 1) * 8
    glvs = [[gl[e, pl.ds(k * 16, 16)] for k in range(4)] for e in es]
    xs = [[glvs[i][k] + gr[e, pl.ds(k * 16, 16)] for k in range(4)]
          for i, e in enumerate(es)]
    ps = [[jnp.maximum(x, x * 0.2) * atts[k] for k, x in enumerate(row)]
          for row in xs]
    for sh in (4, 2, 1):
        ps = [[p + _take(p, iota ^ sh) for p in row] for row in ps]
    exs = [[jnp.exp(p) for p in row] for row in ps]
    for i, e in enumerate(es):
        for k in range(4):
            vals[e, pl.ds(k * 16, 16)] = glvs[i][k] * exs[i][k]
    for i, e in enumerate(es):
        dv = jnp.zeros((16,), jnp.float32)
        for k in range(4):
            dv = dv + jnp.where((iota >> 1) == k, _take(exs[i][k], dsel), 0.0)
        vals[e, pl.ds(64, _DEN)] = dv


def _ec_ch40(gl, gr, atts, vals, es):
    iota = lax.iota(jnp.int32, 16)
    glvs = [[gl[e, pl.ds(k * 16, 16)] for k in range(5)] for e in es]
    xs = [[glvs[i][k] + gr[e, pl.ds(k * 16, 16)] for k in range(5)]
          for i, e in enumerate(es)]
    ps = [[jnp.maximum(x, x * 0.2) * atts[k] for k, x in enumerate(row)]
          for row in xs]

    lo = [row[0] + row[1] for row in ps]
    hi = [row[3] + row[4] for row in ps]
    b2 = [row[2] for row in ps]
    for sh in (8, 4, 2, 1):
        lo = [v + _take(v, iota ^ sh) for v in lo]
        hi = [v + _take(v, iota ^ sh) for v in hi]
        if sh != 8:
            b2 = [v + _take(v, iota ^ sh) for v in b2]
    e0s = [l + _take(b, iota & 7) for l, b in zip(lo, b2)]
    e1s = [_take(b, iota | 8) + h for h, b in zip(hi, b2)]
    ex0s = [jnp.exp(v) for v in e0s]
    ex1s = [jnp.exp(v) for v in e1s]
    for i, e in enumerate(es):
        exm = jnp.where(iota < 8, ex0s[i], ex1s[i])
        for k, fac in enumerate((ex0s[i], ex0s[i], exm, ex1s[i], ex1s[i])):
            vals[e, pl.ds(k * 16, 16)] = glvs[i][k] * fac
    for i, e in enumerate(es):
        dv = jnp.where(iota == 0, ex0s[i],
                       jnp.where(iota == 1, ex1s[i], 0.0))
        vals[e, pl.ds(80, _DEN)] = dv


def _edge_body(ch, xl_hbm, xr_hbm, src_hbm, dst_hbm, att_hbm, z_hbm, out_hbm,
               att_v, *rest):
    gls = rest[0:_NBUF]
    grs = rest[_NBUF:2 * _NBUF]
    vls = rest[2 * _NBUF:3 * _NBUF]
    ixs = rest[3 * _NBUF:4 * _NBUF]
    ixd = rest[4 * _NBUF:5 * _NBUF]
    isems = rest[5 * _NBUF:6 * _NBUF]
    gsems = rest[6 * _NBUF:7 * _NBUF]
    ssems = rest[7 * _NBUF:8 * _NBUF]
    acc_sp = rest[8 * _NBUF]
    compute = _ec_ch8 if ch == 8 else _ec_ch40

    cid = lax.axis_index("c")
    sid = lax.axis_index("s")
    wid = cid * _NSUB + sid

    pltpu.sync_copy(att_hbm, att_v)
    atts = tuple(att_v[pl.ds(k * 16, 16)]
                 for k in range((64 if ch == 8 else 80) // 16))

    pltpu.sync_copy(z_hbm, vls[0])

    def zc(i, _):
        chn = sid + i * _NSUB

        @pl.when(chn < _N // _C)
        def _():
            pltpu.sync_copy(vls[0], acc_sp.at[pl.ds(chn * _C, _C)])
        return 0

    lax.fori_loop(0, (_N // _C + _NSUB - 1) // _NSUB, zc, 0)
    plsc.subcore_barrier()

    def fire_i(j, b):
        pltpu.async_copy(src_hbm.at[wid, j], ixs[b], isems[b])
        pltpu.async_copy(dst_hbm.at[wid, j], ixd[b], isems[b])

    def drain_i(j, b):
        pltpu.make_async_copy(src_hbm.at[wid, j], ixs[b], isems[b]).wait()
        pltpu.make_async_copy(dst_hbm.at[wid, j], ixd[b], isems[b]).wait()

    def fire_g(j, b):
        pltpu.async_copy(xl_hbm.at[ixs[b]], gls[b], gsems[b])
        pltpu.async_copy(xr_hbm.at[ixd[b]], grs[b], gsems[b])

    def drain_g(j, b):
        pltpu.make_async_copy(xl_hbm.at[ixs[b]], gls[b], gsems[b]).wait()
        pltpu.make_async_copy(xr_hbm.at[ixd[b]], grs[b], gsems[b]).wait()

    def fire_s(j, b):
        pltpu.async_copy(vls[b], acc_sp.at[ixd[b]], ssems[b], add=True)

    def drain_s(j, b):
        pltpu.make_async_copy(vls[b], acc_sp.at[ixd[b]], ssems[b]).wait()

    for j in range(3):
        fire_i(j, j % _NBUF)
    for j in range(2):
        drain_i(j, j % _NBUF)
        fire_g(j, j % _NBUF)

    def outer(o, _):
        j0 = o * _UNROLL
        for u in range(_UNROLL):
            j = j0 + u
            b = u % _NBUF

            @pl.when(j >= 2)
            def _(j=j, b=b):
                drain_s(j - 2, (b + 3) % _NBUF)

            @pl.when(j + 3 < _CPW)
            def _(j=j, b=b):
                fire_i(j + 3, (b + 3) % _NBUF)

            @pl.when(j + 2 < _CPW)
            def _(j=j, b=b):
                drain_i(j + 2, (b + 2) % _NBUF)
                fire_g(j + 2, (b + 2) % _NBUF)

            drain_g(j, b)

            ne = 8 if ch == 8 else 4

            def ec(i, _, b=b, ne=ne):
                compute(gls[b], grs[b], atts, vls[b],
                        tuple(i * ne + v for v in range(ne)))
                return 0

            lax.fori_loop(0, _C // ne, ec, 0)
            fire_s(j, b)
        return 0

    lax.fori_loop(0, _CPW // _UNROLL, outer, 0)
    for j in range(_CPW - 2, _CPW):
        drain_s(j, j % _NBUF)
    plsc.subcore_barrier()

    def dc(i, _):
        chn = sid + i * _NSUB

        @pl.when(chn < _N // _C)
        def _():
            pltpu.sync_copy(acc_sp.at[pl.ds(chn * _C, _C)], vls[0])
            pltpu.sync_copy(
                vls[0], out_hbm.at[pl.ds(cid * _N + chn * _C, _C)])
        return 0

    lax.fori_loop(0, (_N // _C + _NSUB - 1) // _NSUB, dc, 0)


def _edge_sc(xl, xr, attv, src3, dst3, zrows):
    w = xl.shape[1]
    ch = 8 if w == 64 else 40
    f = w + _DEN
    mesh = plsc.VectorSubcoreMesh(core_axis_name="c", subcore_axis_name="s")
    out = pl.kernel(
        functools.partial(_edge_body, ch),
        out_type=jax.ShapeDtypeStruct((_NCORE * _N, f), jnp.float32),
        mesh=mesh,
        compiler_params=pltpu.CompilerParams(use_tc_tiling_on_sc=False),
        scratch_types=(
            [pltpu.VMEM((w,), jnp.float32)]
            + [pltpu.VMEM((_C, w), jnp.float32) for _ in range(2 * _NBUF)]
            + [pltpu.VMEM((_C, f), jnp.float32) for _ in range(_NBUF)]
            + [pltpu.VMEM((_C,), jnp.int32) for _ in range(2 * _NBUF)]
            + [pltpu.SemaphoreType.DMA for _ in range(3 * _NBUF)]
            + [pltpu.VMEM_SHARED((_N, f), jnp.float32)]),
    )(xl, xr, src3, dst3, attv, zrows)
    return out.reshape(_NCORE, _N, f)


def _att_mats(att):
    h, ch = att.shape
    A = (att[:, :, None] * jnp.eye(h, dtype=jnp.float32)[:, None, :]).reshape(
        h * ch, h)
    K = jnp.kron(jnp.eye(h, dtype=jnp.float32),
                 jnp.ones((1, ch), jnp.float32))
    return A, K


def kernel(x, edge_index, Wl1, Wr1, att1, b1, Wl2, Wr2, att2, b2, Wl3, Wr3,
           att3, b3, Wl4, Wr4, att4, b4, Wl5, Wr5, att5, b5, R1, rb1, R2,
           rb2, R3, rb3, R4, rb4):
    src3 = edge_index[0].reshape(_NW, _CPW, _C)
    dst3 = edge_index[1].reshape(_NW, _CPW, _C)
    z80 = jnp.zeros((_C, 64 + _DEN), jnp.float32)
    z96 = jnp.zeros((_C, 80 + _DEN), jnp.float32)

    A1, K1 = _att_mats(att1)
    S5 = jnp.tile(jnp.eye(_NC, dtype=jnp.float32), (2, 1))

    xl, xr = _lin(x, Wl1, Wr1)
    acc = _edge_sc(xl, xr, att1.reshape(-1), src3, dst3, z80)
    h = _mid(acc, xl, xr, x, A1, K1, b1.reshape(1, -1), R1, rb1.reshape(1, -1))

    for (Wl, Wr, att, b, R, rb) in (
            (Wl2, Wr2, att2, b2, R2, rb2),
            (Wl3, Wr3, att3, b3, R3, rb3),
            (Wl4, Wr4, att4, b4, R4, rb4)):
        A, K = _att_mats(att)
        xl, xr = _lin(h, Wl, Wr)
        acc = _edge_sc(xl, xr, att.reshape(-1), src3, dst3, z80)
        h = _mid(acc, xl, xr, h, A, K, b.reshape(1, -1), R, rb.reshape(1, -1))

    accs, xls, xrs, As = [], [], [], []
    K5 = None
    for k in range(4):
        wsl = slice(k * 80, (k + 1) * 80)
        xlk, xrk = _lin(h, Wl5[:, wsl], Wr5[:, wsl])
        attk = att5[2 * k:2 * k + 2]
        A5k, K5 = _att_mats(attk)
        acck = _edge_sc(xlk, xrk, attk.reshape(-1), src3, dst3, z96)
        accs.append(acck)
        xls.append(xlk)
        xrs.append(xrk)
        As.append(A5k)
    return _final(accs, xls, xrs, As, K5, S5, b5.reshape(1, -1))


# speedup vs baseline: 1.0545x; 1.0545x over previous
import functools

import jax
import jax.numpy as jnp
from jax import lax
from jax.experimental import pallas as pl
from jax.experimental.pallas import tpu as pltpu
from jax.experimental.pallas import tpu_sc as plsc

_N = 10000
_E = 320000
_H = 8
_HID = 8
_NC = 40
_BLK = 2000

_NCORE = 2
_NSUB = 16
_NW = _NCORE * _NSUB
_EW = _E // _NW
_C = 40
_CPW = _EW // _C
_NBUF = 5
_UNROLL = 10
_DEN = 16


def _leaky(v):
    return jnp.where(v >= 0, v, 0.2 * v)


def _dot(a, b):
    return jnp.dot(a, b, preferred_element_type=jnp.float32)


def _lin_body(h_ref, wl_ref, wr_ref, xl_ref, xr_ref):
    h = h_ref[...]
    xl_ref[...] = _dot(h, wl_ref[...])
    xr_ref[...] = _dot(h, wr_ref[...])


def _lin(h, Wl, Wr):
    n, din = h.shape
    w = Wl.shape[1]
    return pl.pallas_call(
        _lin_body,
        grid=(n // _BLK,),
        in_specs=[
            pl.BlockSpec((_BLK, din), lambda i: (i, 0)),
            pl.BlockSpec((din, w), lambda i: (0, 0)),
            pl.BlockSpec((din, w), lambda i: (0, 0)),
        ],
        out_specs=[
            pl.BlockSpec((_BLK, w), lambda i: (i, 0)),
            pl.BlockSpec((_BLK, w), lambda i: (i, 0)),
        ],
        out_shape=[jax.ShapeDtypeStruct((n, w), jnp.float32)] * 2,
    )(h, Wl, Wr)


def _mid_body(acc_ref, xl_ref, xr_ref, hprev_ref, a_ref, k_ref, b_ref,
              r_ref, rb_ref, h_ref):
    xl = xl_ref[...]
    xr = xr_ref[...]
    w = xl.shape[1]
    exs = jnp.exp(_dot(_leaky(xl + xr), a_ref[...]))
    acc = acc_ref[0] + acc_ref[1]
    num = acc[:, :w] + _dot(exs, k_ref[...]) * xl
    den = acc[:, w:w + _H] + exs
    denx = _dot(den, k_ref[...]) + 1e-16
    gat = num / denx + b_ref[...]
    h_ref[...] = _leaky(gat + _dot(hprev_ref[...], r_ref[...]) + rb_ref[...])


def _mid(acc, xl, xr, hprev, A, K, b, R, rb):
    n, w = xl.shape
    din = hprev.shape[1]
    f = w + _DEN
    return pl.pallas_call(
        _mid_body,
        grid=(n // _BLK,),
        in_specs=[
            pl.BlockSpec((2, _BLK, f), lambda i: (0, i, 0)),
            pl.BlockSpec((_BLK, w), lambda i: (i, 0)),
            pl.BlockSpec((_BLK, w), lambda i: (i, 0)),
            pl.BlockSpec((_BLK, din), lambda i: (i, 0)),
            pl.BlockSpec((w, _H), lambda i: (0, 0)),
            pl.BlockSpec((_H, w), lambda i: (0, 0)),
            pl.BlockSpec((1, w), lambda i: (0, 0)),
            pl.BlockSpec((din, w), lambda i: (0, 0)),
            pl.BlockSpec((1, w), lambda i: (0, 0)),
        ],
        out_specs=pl.BlockSpec((_BLK, w), lambda i: (i, 0)),
        out_shape=jax.ShapeDtypeStruct((n, w), jnp.float32),
    )(acc, xl, xr, hprev, A, K, b, R, rb)


def _final_body(*refs):
    accs = refs[0:4]
    xls = refs[4:8]
    xrs = refs[8:12]
    aas = refs[12:16]
    k_ref, s_ref, b_ref, out_ref = refs[16:]
    w = xls[0].shape[1]

    tot = None
    for acc_ref, xl_ref, xr_ref, a_ref in zip(accs, xls, xrs, aas):
        xl = xl_ref[...]
        exs = jnp.exp(_dot(_leaky(xl + xr_ref[...]), a_ref[...]))
        acc = acc_ref[0] + acc_ref[1]
        num = acc[:, :w] + _dot(exs, k_ref[...]) * xl
        den = acc[:, w:w + 2] + exs
        gat = num / (_dot(den, k_ref[...]) + 1e-16)
        part = _dot(gat, s_ref[...])
        tot = part if tot is None else tot + part
    out = tot * 0.125 + b_ref[...]
    mx = jnp.max(out, axis=1, keepdims=True)
    lse = jnp.log(jnp.sum(jnp.exp(out - mx), axis=1, keepdims=True)) + mx
    out_ref[...] = out - lse


def _final(accs, xls, xrs, As, K, S, b5):
    n, w = xls[0].shape
    f = w + _DEN
    acc_spec = pl.BlockSpec((2, _BLK, f), lambda i: (0, i, 0))
    row_spec = pl.BlockSpec((_BLK, w), lambda i: (i, 0))
    a_spec = pl.BlockSpec((w, 2), lambda i: (0, 0))
    return pl.pallas_call(
        _final_body,
        grid=(n // _BLK,),
        in_specs=([acc_spec] * 4 + [row_spec] * 8 + [a_spec] * 4 + [
            pl.BlockSpec((2, w), lambda i: (0, 0)),
            pl.BlockSpec((w, _NC), lambda i: (0, 0)),
            pl.BlockSpec((1, _NC), lambda i: (0, 0)),
        ]),
        out_specs=pl.BlockSpec((_BLK, _NC), lambda i: (i, 0)),
        out_shape=jax.ShapeDtypeStruct((n, _NC), jnp.float32),
    )(*accs, *xls, *xrs, *As, K, S, b5)


_GD = lax.GatherDimensionNumbers(
    offset_dims=(), collapsed_slice_dims=(0,), start_index_map=(0,))


def _take(v, idx):
    return lax.gather(v, idx[:, None], _GD, (1,),
                      mode=lax.GatherScatterMode.PROMISE_IN_BOUNDS)


def _ec_ch8(gl, gr, atts, vals, es):
    iota = lax.iota(jnp.int32, 16)
    dsel = (iota & 1) * 8
    glvs = [[gl[e, pl.ds(k * 16, 16)] for k in range(4)] for e in es]
    xs = [[glvs[i][k] + gr[e, pl.ds(k * 16, 16)] for k in range(4)]
          for i, e in enumerate(es)]
    ps = [[jnp.maximum(x, x * 0.2) * atts[k] for k, x in enumerate(row)]
          for row in xs]
    for sh in (4, 2, 1):
        ps = [[p + _take(p, iota ^ sh) for p in row] for row in ps]
    exs = [[jnp.exp(p) for p in row] for row in ps]
    for i, e in enumerate(es):
        for k in range(4):
            vals[e, pl.ds(k * 16, 16)] = glvs[i][k] * exs[i][k]
    for i, e in enumerate(es):
        dv = jnp.zeros((16,), jnp.float32)
        for k in range(4):
            dv = dv + jnp.where((iota >> 1) == k, _take(exs[i][k], dsel), 0.0)
        vals[e, pl.ds(64, _DEN)] = dv


def _ec_ch40(gl, gr, atts, vals, es):
    iota = lax.iota(jnp.int32, 16)
    glvs = [[gl[e, pl.ds(k * 16, 16)] for k in range(5)] for e in es]
    xs = [[glvs[i][k] + gr[e, pl.ds(k * 16, 16)] for k in range(5)]
          for i, e in enumerate(es)]
    ps = [[jnp.maximum(x, x * 0.2) * atts[k] for k, x in enumerate(row)]
          for row in xs]

    lo = [row[0] + row[1] for row in ps]
    hi = [row[3] + row[4] for row in ps]
    b2 = [row[2] for row in ps]
    for sh in (8, 4, 2, 1):
        lo = [v + _take(v, iota ^ sh) for v in lo]
        hi = [v + _take(v, iota ^ sh) for v in hi]
        if sh != 8:
            b2 = [v + _take(v, iota ^ sh) for v in b2]
    e0s = [l + _take(b, iota & 7) for l, b in zip(lo, b2)]
    e1s = [_take(b, iota | 8) + h for h, b in zip(hi, b2)]
    ex0s = [jnp.exp(v) for v in e0s]
    ex1s = [jnp.exp(v) for v in e1s]
    for i, e in enumerate(es):
        exm = jnp.where(iota < 8, ex0s[i], ex1s[i])
        for k, fac in enumerate((ex0s[i], ex0s[i], exm, ex1s[i], ex1s[i])):
            vals[e, pl.ds(k * 16, 16)] = glvs[i][k] * fac
    for i, e in enumerate(es):
        dv = jnp.where(iota == 0, ex0s[i],
                       jnp.where(iota == 1, ex1s[i], 0.0))
        vals[e, pl.ds(80, _DEN)] = dv


def _edge_body(ch, xl_hbm, xr_hbm, src_hbm, dst_hbm, att_hbm, z_hbm, out_hbm,
               att_v, *rest):
    gls = rest[0:_NBUF]
    grs = rest[_NBUF:2 * _NBUF]
    vls = rest[2 * _NBUF:3 * _NBUF]
    ixs = rest[3 * _NBUF:4 * _NBUF]
    ixd = rest[4 * _NBUF:5 * _NBUF]
    isems = rest[5 * _NBUF:6 * _NBUF]
    gsems = rest[6 * _NBUF:7 * _NBUF]
    ssems = rest[7 * _NBUF:8 * _NBUF]
    acc_sp = rest[8 * _NBUF]
    compute = _ec_ch8 if ch == 8 else _ec_ch40

    cid = lax.axis_index("c")
    sid = lax.axis_index("s")
    wid = cid * _NSUB + sid

    pltpu.sync_copy(att_hbm, att_v)
    atts = tuple(att_v[pl.ds(k * 16, 16)]
                 for k in range((64 if ch == 8 else 80) // 16))

    pltpu.sync_copy(z_hbm, vls[0])

    def zc(i, _):
        chn = sid + i * _NSUB

        @pl.when(chn < _N // _C)
        def _():
            pltpu.sync_copy(vls[0], acc_sp.at[pl.ds(chn * _C, _C)])
        return 0

    lax.fori_loop(0, (_N // _C + _NSUB - 1) // _NSUB, zc, 0)
    plsc.subcore_barrier()

    def fire_i(j, b):
        pltpu.async_copy(src_hbm.at[wid, j], ixs[b], isems[b])
        pltpu.async_copy(dst_hbm.at[wid, j], ixd[b], isems[b])

    def drain_i(j, b):
        pltpu.make_async_copy(src_hbm.at[wid, j], ixs[b], isems[b]).wait()
        pltpu.make_async_copy(dst_hbm.at[wid, j], ixd[b], isems[b]).wait()

    def fire_g(j, b):
        pltpu.async_copy(xl_hbm.at[ixs[b]], gls[b], gsems[b])
        pltpu.async_copy(xr_hbm.at[ixd[b]], grs[b], gsems[b])

    def drain_g(j, b):
        pltpu.make_async_copy(xl_hbm.at[ixs[b]], gls[b], gsems[b]).wait()
        pltpu.make_async_copy(xr_hbm.at[ixd[b]], grs[b], gsems[b]).wait()

    def fire_s(j, b):
        pltpu.async_copy(vls[b], acc_sp.at[ixd[b]], ssems[b], add=True)

    def drain_s(j, b):
        pltpu.make_async_copy(vls[b], acc_sp.at[ixd[b]], ssems[b]).wait()

    for j in range(3):
        fire_i(j, j % _NBUF)
    for j in range(2):
        drain_i(j, j % _NBUF)
        fire_g(j, j % _NBUF)

    def outer(o, _):
        j0 = o * _UNROLL
        for u in range(_UNROLL):
            j = j0 + u
            b = u % _NBUF

            @pl.when(j >= 2)
            def _(j=j, b=b):
                drain_s(j - 2, (b + 3) % _NBUF)

            @pl.when(j + 3 < _CPW)
            def _(j=j, b=b):
                fire_i(j + 3, (b + 3) % _NBUF)

            @pl.when(j + 2 < _CPW)
            def _(j=j, b=b):
                drain_i(j + 2, (b + 2) % _NBUF)
                fire_g(j + 2, (b + 2) % _NBUF)

            drain_g(j, b)

            def ec(i, _, b=b):
                compute(gls[b], grs[b], atts, vls[b],
                        tuple(i * 4 + v for v in range(4)))
                return 0

            lax.fori_loop(0, _C // 4, ec, 0)
            fire_s(j, b)
        return 0

    lax.fori_loop(0, _CPW // _UNROLL, outer, 0)
    for j in range(_CPW - 2, _CPW):
        drain_s(j, j % _NBUF)
    plsc.subcore_barrier()

    def dc(i, _):
        chn = sid + i * _NSUB

        @pl.when(chn < _N // _C)
        def _():
            pltpu.sync_copy(acc_sp.at[pl.ds(chn * _C, _C)], vls[0])
            pltpu.sync_copy(
                vls[0], out_hbm.at[pl.ds(cid * _N + chn * _C, _C)])
        return 0

    lax.fori_loop(0, (_N // _C + _NSUB - 1) // _NSUB, dc, 0)


def _edge_sc(xl, xr, attv, src3, dst3, zrows):
    w = xl.shape[1]
    ch = 8 if w == 64 else 40
    f = w + _DEN
    mesh = plsc.VectorSubcoreMesh(core_axis_name="c", subcore_axis_name="s")
    out = pl.kernel(
        functools.partial(_edge_body, ch),
        out_type=jax.ShapeDtypeStruct((_NCORE * _N, f), jnp.float32),
        mesh=mesh,
        compiler_params=pltpu.CompilerParams(use_tc_tiling_on_sc=False),
        scratch_types=(
            [pltpu.VMEM((w,), jnp.float32)]
            + [pltpu.VMEM((_C, w), jnp.float32) for _ in range(2 * _NBUF)]
            + [pltpu.VMEM((_C, f), jnp.float32) for _ in range(_NBUF)]
            + [pltpu.VMEM((_C,), jnp.int32) for _ in range(2 * _NBUF)]
            + [pltpu.SemaphoreType.DMA for _ in range(3 * _NBUF)]
            + [pltpu.VMEM_SHARED((_N, f), jnp.float32)]),
    )(xl, xr, src3, dst3, attv, zrows)
    return out.reshape(_NCORE, _N, f)


def _att_mats(att):
    h, ch = att.shape
    A = (att[:, :, None] * jnp.eye(h, dtype=jnp.float32)[:, None, :]).reshape(
        h * ch, h)
    K = jnp.kron(jnp.eye(h, dtype=jnp.float32),
                 jnp.ones((1, ch), jnp.float32))
    return A, K


def kernel(x, edge_index, Wl1, Wr1, att1, b1, Wl2, Wr2, att2, b2, Wl3, Wr3,
           att3, b3, Wl4, Wr4, att4, b4, Wl5, Wr5, att5, b5, R1, rb1, R2,
           rb2, R3, rb3, R4, rb4):
    src3 = edge_index[0].reshape(_NW, _CPW, _C)
    dst3 = edge_index[1].reshape(_NW, _CPW, _C)
    z80 = jnp.zeros((_C, 64 + _DEN), jnp.float32)
    z96 = jnp.zeros((_C, 80 + _DEN), jnp.float32)

    A1, K1 = _att_mats(att1)
    S5 = jnp.tile(jnp.eye(_NC, dtype=jnp.float32), (2, 1))

    xl, xr = _lin(x, Wl1, Wr1)
    acc = _edge_sc(xl, xr, att1.reshape(-1), src3, dst3, z80)
    h = _mid(acc, xl, xr, x, A1, K1, b1.reshape(1, -1), R1, rb1.reshape(1, -1))

    for (Wl, Wr, att, b, R, rb) in (
            (Wl2, Wr2, att2, b2, R2, rb2),
            (Wl3, Wr3, att3, b3, R3, rb3),
            (Wl4, Wr4, att4, b4, R4, rb4)):
        A, K = _att_mats(att)
        xl, xr = _lin(h, Wl, Wr)
        acc = _edge_sc(xl, xr, att.reshape(-1), src3, dst3, z80)
        h = _mid(acc, xl, xr, h, A, K, b.reshape(1, -1), R, rb.reshape(1, -1))

    accs, xls, xrs, As = [], [], [], []
    K5 = None
    for k in range(4):
        wsl = slice(k * 80, (k + 1) * 80)
        xlk, xrk = _lin(h, Wl5[:, wsl], Wr5[:, wsl])
        attk = att5[2 * k:2 * k + 2]
        A5k, K5 = _att_mats(attk)
        acck = _edge_sc(xlk, xrk, attk.reshape(-1), src3, dst3, z96)
        accs.append(acck)
        xls.append(xlk)
        xrs.append(xrk)
        As.append(A5k)
    return _final(accs, xls, xrs, As, K5, S5, b5.reshape(1, -1))
